# baseline (speedup 1.0000x reference)
import functools

import jax
import jax.numpy as jnp
from jax import lax
from jax.experimental import pallas as pl
from jax.experimental.pallas import tpu as pltpu
from jax.experimental.pallas import tpu_sc as plsc

VOCAB = 1000000
EMBED_DIM = 64

_B = 4096 * 50
_CHUNK = 128
_NW = 32
_ROWS_PER_W = _B // _NW
_CHUNKS_PER_W = _ROWS_PER_W // _CHUNK

_BC = 4096
_H = 507904


def _repack_kernel(wta_ref, wtb_ref, out_ref):
    out_ref[...] = jnp.concatenate([wta_ref[...].T, wtb_ref[...].T], axis=1)


def _repack(wt):
    grid = _H // _BC
    return pl.pallas_call(
        _repack_kernel,
        grid=(grid,),
        in_specs=[
            pl.BlockSpec((EMBED_DIM, _BC), lambda i: (0, i)),
            pl.BlockSpec(
                (EMBED_DIM, _BC),
                lambda i: (0, jnp.minimum(i + _H // _BC, (VOCAB - 1) // _BC)),
            ),
        ],
        out_specs=pl.BlockSpec((_BC, 128), lambda i: (i, 0)),
        out_shape=jax.ShapeDtypeStruct((_H, 128), jnp.float32),
    )(wt, wt)


def _gather_kernel(table_hbm, idx_hbm, out_hbm,
                   idx_v, h_v, rows_a, rows_b, comp_a, comp_b, sem_a, sem_b):
    wid = lax.axis_index("s") * 2 + lax.axis_index("c")
    out0 = wid * _ROWS_PER_W

    pltpu.sync_copy(idx_hbm.at[wid], idx_v)

    def prep(k, _):
        j = k // 8
        g = k % 8
        v = idx_v[j, pl.ds(g * 16, 16)]
        m = v >= _H
        idx_v[j, pl.ds(g * 16, 16)] = jnp.where(m, v - _H, v)
        h_v[j, pl.ds(g * 16, 16)] = jnp.where(m, EMBED_DIM, 0)
        return 0

    lax.fori_loop(0, _CHUNKS_PER_W * (_CHUNK // 16), prep, 0)

    def select(j, rows, comp):
        def grp(g, _):
            rvec = lax.iota(jnp.int32, 16) + g * 16
            hvec = h_v[j, pl.ds(g * 16, 16)]
            for d in range(EMBED_DIM):
                x = plsc.load_gather(rows, [rvec, hvec + d])
                plsc.store_scatter(comp, [rvec, jnp.full((16,), d, jnp.int32)], x)
            return 0
        lax.fori_loop(0, _CHUNK // 16, grp, 0)

    pltpu.async_copy(table_hbm.at[idx_v.at[0]], rows_a, sem_a)

    def body(j, _):
        pltpu.async_copy(table_hbm.at[idx_v.at[j + 1]], rows_b, sem_b)
        pltpu.make_async_copy(table_hbm.at[idx_v.at[j]], rows_a, sem_a).wait()
        select(j, rows_a, comp_a)
        pltpu.sync_copy(comp_a, out_hbm.at[pl.ds(out0 + j * _CHUNK, _CHUNK)])
        nxt = jnp.minimum(j + 2, _CHUNKS_PER_W - 1)
        pltpu.async_copy(table_hbm.at[idx_v.at[nxt]], rows_a, sem_a)
        pltpu.make_async_copy(table_hbm.at[idx_v.at[j + 1]], rows_b, sem_b).wait()
        select(j + 1, rows_b, comp_b)
        pltpu.sync_copy(comp_b, out_hbm.at[pl.ds(out0 + (j + 1) * _CHUNK, _CHUNK)])
        return 0

    lax.fori_loop(0, _CHUNKS_PER_W // 2, lambda i, c: body(i * 2, c), 0)
    pltpu.make_async_copy(table_hbm.at[idx_v.at[_CHUNKS_PER_W - 1]], rows_a, sem_a).wait()


def _gather(table, idx3d):
    mesh = plsc.VectorSubcoreMesh(core_axis_name="c", subcore_axis_name="s")
    f = functools.partial(
        pl.kernel,
        out_type=jax.ShapeDtypeStruct((_B, 128), jnp.float32),
        mesh=mesh,
        scratch_types=[
            pltpu.VMEM((_CHUNKS_PER_W, _CHUNK), jnp.int32),
            pltpu.VMEM((_CHUNKS_PER_W, _CHUNK), jnp.int32),
            pltpu.VMEM((_CHUNK, 128), jnp.float32),
            pltpu.VMEM((_CHUNK, 128), jnp.float32),
            pltpu.VMEM((_CHUNK, 128), jnp.float32),
            pltpu.VMEM((_CHUNK, 128), jnp.float32),
            pltpu.SemaphoreType.DMA,
            pltpu.SemaphoreType.DMA,
        ],
        compiler_params=pltpu.CompilerParams(
            use_tc_tiling_on_sc=True, needs_layout_passes=False
        ),
    )(_gather_kernel)
    return f(table, idx3d)


def kernel(input_, weight):
    table = _repack(weight.T)
    idx3d = input_.reshape(_NW, _CHUNKS_PER_W, _CHUNK)
    out = _gather(table, idx3d)
    return out[:, :EMBED_DIM].reshape(input_.shape + (EMBED_DIM,))


# speedup vs baseline: 1.8224x; 1.8224x over previous
import functools

import jax
import jax.numpy as jnp
from jax import lax
from jax.experimental import pallas as pl
from jax.experimental.pallas import tpu as pltpu
from jax.experimental.pallas import tpu_sc as plsc

VOCAB = 1000000
EMBED_DIM = 64

_B = 4096 * 50
_CHUNK = 128
_NW = 32
_ROWS_PER_W = _B // _NW
_CHUNKS_PER_W = _ROWS_PER_W // _CHUNK

_BC = 8192


def _repack_kernel(wt_ref, out_ref):
    out_ref[:, 0:EMBED_DIM] = wt_ref[...].T


def _repack(wt):
    grid = (VOCAB + _BC - 1) // _BC
    return pl.pallas_call(
        _repack_kernel,
        grid=(grid,),
        in_specs=[pl.BlockSpec((EMBED_DIM, _BC), lambda i: (0, i))],
        out_specs=pl.BlockSpec((_BC, 128), lambda i: (i, 0)),
        out_shape=jax.ShapeDtypeStruct((VOCAB, 128), jnp.float32),
    )(wt)


def _gather_kernel(table_hbm, idx_hbm, out_hbm, idx_v, rows_a, rows_b, sem_a, sem_b):
    wid = lax.axis_index("s") * 2 + lax.axis_index("c")
    out0 = wid * _ROWS_PER_W

    pltpu.sync_copy(idx_hbm.at[wid], idx_v)

    pltpu.async_copy(table_hbm.at[idx_v.at[0]], rows_a, sem_a)

    def body(j, _):
        pltpu.async_copy(table_hbm.at[idx_v.at[j + 1]], rows_b, sem_b)
        pltpu.make_async_copy(table_hbm.at[idx_v.at[j]], rows_a, sem_a).wait()
        pltpu.sync_copy(rows_a, out_hbm.at[pl.ds(out0 + j * _CHUNK, _CHUNK)])
        nxt = jnp.minimum(j + 2, _CHUNKS_PER_W - 1)
        pltpu.async_copy(table_hbm.at[idx_v.at[nxt]], rows_a, sem_a)
        pltpu.make_async_copy(table_hbm.at[idx_v.at[j + 1]], rows_b, sem_b).wait()
        pltpu.sync_copy(rows_b, out_hbm.at[pl.ds(out0 + (j + 1) * _CHUNK, _CHUNK)])
        return 0

    lax.fori_loop(0, _CHUNKS_PER_W // 2, lambda i, c: body(i * 2, c), 0)
    pltpu.make_async_copy(table_hbm.at[idx_v.at[_CHUNKS_PER_W - 1]], rows_a, sem_a).wait()


def _gather(table, idx3d):
    mesh = plsc.VectorSubcoreMesh(core_axis_name="c", subcore_axis_name="s")
    f = functools.partial(
        pl.kernel,
        out_type=jax.ShapeDtypeStruct((_B, 128), jnp.float32),
        mesh=mesh,
        scratch_types=[
            pltpu.VMEM((_CHUNKS_PER_W, _CHUNK), jnp.int32),
            pltpu.VMEM((_CHUNK, 128), jnp.float32),
            pltpu.VMEM((_CHUNK, 128), jnp.float32),
            pltpu.SemaphoreType.DMA,
            pltpu.SemaphoreType.DMA,
        ],
        compiler_params=pltpu.CompilerParams(use_tc_tiling_on_sc=True),
    )(_gather_kernel)
    return f(table, idx3d)


def kernel(input_, weight):
    table = _repack(weight.T)
    idx3d = input_.reshape(_NW, _CHUNKS_PER_W, _CHUNK)
    out = _gather(table, idx3d)
    return out[:, :EMBED_DIM].reshape(input_.shape + (EMBED_DIM,))


# speedup vs baseline: 1.8927x; 1.0386x over previous
import functools

import jax
import jax.numpy as jnp
from jax import lax
from jax.experimental import pallas as pl
from jax.experimental.pallas import tpu as pltpu
from jax.experimental.pallas import tpu_sc as plsc

VOCAB = 1000000
EMBED_DIM = 64

_B = 4096 * 50
_CHUNK = 128
_NW = 32
_ROWS_PER_W = _B // _NW
_CHUNKS_PER_W = _ROWS_PER_W // _CHUNK

_BC = 16384


def _repack_kernel(wt_ref, out_ref):
    out_ref[:, 0:EMBED_DIM] = wt_ref[...].T


def _repack(wt):
    grid = (VOCAB + _BC - 1) // _BC
    return pl.pallas_call(
        _repack_kernel,
        grid=(grid,),
        in_specs=[pl.BlockSpec((EMBED_DIM, _BC), lambda i: (0, i))],
        out_specs=pl.BlockSpec((_BC, 128), lambda i: (i, 0)),
        out_shape=jax.ShapeDtypeStruct((VOCAB, 128), jnp.float32),
    )(wt)


def _gather_kernel(table_hbm, idx_hbm, out_hbm, idx_v, rows_a, rows_b, sem_a, sem_b):
    wid = lax.axis_index("s") * 2 + lax.axis_index("c")
    out0 = wid * _ROWS_PER_W

    pltpu.sync_copy(idx_hbm.at[wid], idx_v)

    pltpu.async_copy(table_hbm.at[idx_v.at[0]], rows_a, sem_a)

    def body(j, _):
        pltpu.async_copy(table_hbm.at[idx_v.at[j + 1]], rows_b, sem_b)
        pltpu.make_async_copy(table_hbm.at[idx_v.at[j]], rows_a, sem_a).wait()
        pltpu.sync_copy(rows_a, out_hbm.at[pl.ds(out0 + j * _CHUNK, _CHUNK)])
        nxt = jnp.minimum(j + 2, _CHUNKS_PER_W - 1)
        pltpu.async_copy(table_hbm.at[idx_v.at[nxt]], rows_a, sem_a)
        pltpu.make_async_copy(table_hbm.at[idx_v.at[j + 1]], rows_b, sem_b).wait()
        pltpu.sync_copy(rows_b, out_hbm.at[pl.ds(out0 + (j + 1) * _CHUNK, _CHUNK)])
        return 0

    lax.fori_loop(0, _CHUNKS_PER_W // 2, lambda i, c: body(i * 2, c), 0)
    pltpu.make_async_copy(table_hbm.at[idx_v.at[_CHUNKS_PER_W - 1]], rows_a, sem_a).wait()


def _gather(table, idx3d):
    mesh = plsc.VectorSubcoreMesh(core_axis_name="c", subcore_axis_name="s")
    f = functools.partial(
        pl.kernel,
        out_type=jax.ShapeDtypeStruct((_B, 128), jnp.float32),
        mesh=mesh,
        scratch_types=[
            pltpu.VMEM((_CHUNKS_PER_W, _CHUNK), jnp.int32),
            pltpu.VMEM((_CHUNK, 128), jnp.float32),
            pltpu.VMEM((_CHUNK, 128), jnp.float32),
            pltpu.SemaphoreType.DMA,
            pltpu.SemaphoreType.DMA,
        ],
        compiler_params=pltpu.CompilerParams(use_tc_tiling_on_sc=True),
    )(_gather_kernel)
    return f(table, idx3d)


def kernel(input_, weight):
    table = _repack(weight.T)
    idx3d = input_.reshape(_NW, _CHUNKS_PER_W, _CHUNK)
    out = _gather(table, idx3d)
    return out[:, :EMBED_DIM].reshape(input_.shape + (EMBED_DIM,))


# speedup vs baseline: 1.9122x; 1.0103x over previous
import functools

import jax
import jax.numpy as jnp
from jax import lax
from jax.experimental import pallas as pl
from jax.experimental.pallas import tpu as pltpu
from jax.experimental.pallas import tpu_sc as plsc

VOCAB = 1000000
EMBED_DIM = 64

_B = 4096 * 50
_CHUNK = 128
_NW = 32
_ROWS_PER_W = _B // _NW
_CHUNKS_PER_W = _ROWS_PER_W // _CHUNK

_BC = 25600


def _repack_kernel(wt_ref, out_ref):
    out_ref[:, 0:EMBED_DIM] = wt_ref[...].T


def _repack(wt):
    grid = (VOCAB + _BC - 1) // _BC
    return pl.pallas_call(
        _repack_kernel,
        grid=(grid,),
        in_specs=[pl.BlockSpec((EMBED_DIM, _BC), lambda i: (0, i))],
        out_specs=pl.BlockSpec((_BC, 128), lambda i: (i, 0)),
        out_shape=jax.ShapeDtypeStruct((VOCAB, 128), jnp.float32),
    )(wt)


def _gather_kernel(table_hbm, idx_hbm, out_hbm, idx_v, rows_a, rows_b, sem_a, sem_b):
    wid = lax.axis_index("s") * 2 + lax.axis_index("c")
    out0 = wid * _ROWS_PER_W

    pltpu.sync_copy(idx_hbm.at[wid], idx_v)

    pltpu.async_copy(table_hbm.at[idx_v.at[0]], rows_a, sem_a)

    def body(j, _):
        pltpu.async_copy(table_hbm.at[idx_v.at[j + 1]], rows_b, sem_b)
        pltpu.make_async_copy(table_hbm.at[idx_v.at[j]], rows_a, sem_a).wait()
        pltpu.sync_copy(rows_a, out_hbm.at[pl.ds(out0 + j * _CHUNK, _CHUNK)])
        nxt = jnp.minimum(j + 2, _CHUNKS_PER_W - 1)
        pltpu.async_copy(table_hbm.at[idx_v.at[nxt]], rows_a, sem_a)
        pltpu.make_async_copy(table_hbm.at[idx_v.at[j + 1]], rows_b, sem_b).wait()
        pltpu.sync_copy(rows_b, out_hbm.at[pl.ds(out0 + (j + 1) * _CHUNK, _CHUNK)])
        return 0

    lax.fori_loop(0, _CHUNKS_PER_W // 2, lambda i, c: body(i * 2, c), 0)
    pltpu.make_async_copy(table_hbm.at[idx_v.at[_CHUNKS_PER_W - 1]], rows_a, sem_a).wait()


def _gather(table, idx3d):
    mesh = plsc.VectorSubcoreMesh(core_axis_name="c", subcore_axis_name="s")
    f = functools.partial(
        pl.kernel,
        out_type=jax.ShapeDtypeStruct((_B, 128), jnp.float32),
        mesh=mesh,
        scratch_types=[
            pltpu.VMEM((_CHUNKS_PER_W, _CHUNK), jnp.int32),
            pltpu.VMEM((_CHUNK, 128), jnp.float32),
            pltpu.VMEM((_CHUNK, 128), jnp.float32),
            pltpu.SemaphoreType.DMA,
            pltpu.SemaphoreType.DMA,
        ],
        compiler_params=pltpu.CompilerParams(use_tc_tiling_on_sc=True),
    )(_gather_kernel)
    return f(table, idx3d)


def kernel(input_, weight):
    table = _repack(weight.T)
    idx3d = input_.reshape(_NW, _CHUNKS_PER_W, _CHUNK)
    out = _gather(table, idx3d)
    return out[:, :EMBED_DIM].reshape(input_.shape + (EMBED_DIM,))
